# baseline (speedup 1.0000x reference)
import functools

import jax
import jax.numpy as jnp
from jax import lax
from jax.experimental import pallas as pl
from jax.experimental.pallas import tpu as pltpu
from jax.experimental.pallas import tpu_sc as plsc

_D = 16
_B = 16384
_H = 50
_C = 20
_BETA = 0.5
_N = 1000001

_HP = 56
_CPD = 24
_OP = 32

_NC = 2
_NS = 16
_NW = _NC * _NS
_RPW = _B // _NW

_RBLK = 65536
_SH = _RBLK.bit_length() - 1
_G = (_N + _RBLK - 1) // _RBLK
_WH_LINES = _G * (_RBLK // 8)
_CT_LINES = _G * (_RBLK // 4)


def _sc_body(hrow_hbm, crow_hbm, wh_hbm, wct_hbm,
             out_hbm,
             hg_v, cg_v, out_v,
             hb0, cb0, hb1, cb1,
             hs0, cs0, hs1, cs1):
    wid = lax.axis_index("s") * _NC + lax.axis_index("c")
    base = wid * _RPW

    pltpu.sync_copy(hrow_hbm.at[pl.ds(base * _HP, _RPW * _HP)], hg_v)
    pltpu.sync_copy(crow_hbm.at[pl.ds(base * _CPD, _RPW * _CPD)], cg_v)

    hbufs = (hb0, hb1)
    cbufs = (cb0, cb1)
    hsems = (hs0, hs1)
    csems = (cs0, cs1)

    _HG = _HP + _H
    _CG = _CPD + _C

    def issue(pair, b):
        pltpu.async_copy(
            wh_hbm.at[hg_v.at[pl.ds(pair * 2 * _HP, _HG)]],
            hbufs[b], hsems[b])
        pltpu.async_copy(
            wct_hbm.at[cg_v.at[pl.ds(pair * 2 * _CPD, _CG)]],
            cbufs[b], csems[b])

    def wait(pair, b):
        pltpu.make_async_copy(
            wh_hbm.at[hg_v.at[pl.ds(pair * 2 * _HP, _HG)]],
            hbufs[b], hsems[b]).wait()
        pltpu.make_async_copy(
            wct_hbm.at[cg_v.at[pl.ds(pair * 2 * _CPD, _CG)]],
            cbufs[b], csems[b]).wait()

    lanes = lax.iota(jnp.int32, _D)

    def compute(row, b, ho, co):
        hb = hbufs[b]
        cb = cbufs[b]
        accs = [hb[ho], hb[ho + 1] * _BETA,
                hb[ho + 2] * _BETA ** 2, hb[ho + 3] * _BETA ** 3]
        for h in range(4, _H):
            accs[h % 4] = accs[h % 4] + hb[ho + h] * (_BETA ** h)
        acc = (accs[0] + accs[1]) + (accs[2] + accs[3])
        ctx = [cb[co + c, 0:_D] for c in range(_C)]
        ss = [ctx[0], ctx[1], ctx[2], ctx[3]]
        for c in range(4, _C):
            ss[c % 4] = ss[c % 4] + ctx[c]
        s = (ss[0] + ss[1]) + (ss[2] + ss[3])
        a = acc + s
        lo = jnp.zeros((_D,), jnp.float32)
        hi = jnp.zeros((_D,), jnp.float32)
        for c in range(_C):
            tgt = cb[co + c, _D:2 * _D]
            u = jnp.sum(tgt * (a - ctx[c]))
            if c < _D:
                lo = jnp.where(lanes == c, u, lo)
            else:
                hi = jnp.where(lanes == (c - _D), u, hi)
        out_v[pl.ds(row * _OP, _D)] = lo
        out_v[pl.ds(row * _OP + _D, _D)] = hi

    issue(0, 0)

    def body(i, carry):
        p = i * 2
        for b in range(2):
            pair = p + b
            nxt = pair + 1

            @pl.when(nxt < _RPW // 2)
            def _():
                issue(nxt, 1 - b)

            wait(pair, b)
            compute(pair * 2, b, 0, 0)
            compute(pair * 2 + 1, b, _HP, _CPD)
        return carry

    lax.fori_loop(0, _RPW // 4, body, 0, unroll=False)

    pltpu.sync_copy(out_v, out_hbm.at[pl.ds(base * _OP, _RPW * _OP)])


_sc_utilities = functools.partial(
    pl.kernel,
    out_type=jax.ShapeDtypeStruct((_B * _OP,), jnp.float32),
    mesh=plsc.VectorSubcoreMesh(core_axis_name="c", subcore_axis_name="s"),
    compiler_params=pltpu.CompilerParams(
        needs_layout_passes=False, use_tc_tiling_on_sc=False),
    scratch_types=[
        pltpu.VMEM((_RPW * _HP,), jnp.int32),
        pltpu.VMEM((_RPW * _CPD,), jnp.int32),
        pltpu.VMEM((_RPW * _OP,), jnp.float32),
        pltpu.VMEM((_HP + _H, _D), jnp.float32),
        pltpu.VMEM((_CPD + _C, 2 * _D), jnp.float32),
        pltpu.VMEM((_HP + _H, _D), jnp.float32),
        pltpu.VMEM((_CPD + _C, 2 * _D), jnp.float32),
        pltpu.SemaphoreType.DMA,
        pltpu.SemaphoreType.DMA,
        pltpu.SemaphoreType.DMA,
        pltpu.SemaphoreType.DMA,
    ],
)(_sc_body)


def _mxu_t(x):
    eye = jnp.eye(128, dtype=jnp.float32)
    return lax.dot_general(x, eye, (((0,), (0,)), ((), ())),
                           preferred_element_type=jnp.float32)


def _repack_body(ht_ref, ct_ref, tt_ref, oh_ref, oct_ref):
    xh = ht_ref[...]
    w8 = _RBLK // 8
    out2h = jnp.concatenate(
        [xh[:, k * w8:(k + 1) * w8] for k in range(8)], axis=0)
    oh_ref[...] = _mxu_t(out2h)
    xc = ct_ref[...]
    xt = tt_ref[...]
    w4 = _RBLK // 4
    parts = []
    for k in range(4):
        parts.append(xc[:, k * w4:(k + 1) * w4])
        parts.append(xt[:, k * w4:(k + 1) * w4])
    out2c = jnp.concatenate(parts, axis=0)
    oct_ref[...] = _mxu_t(out2c)


def _repack_tables(Wh, Wc, Wt):
    return pl.pallas_call(
        _repack_body,
        grid=(_G,),
        in_specs=[pl.BlockSpec((_D, _RBLK), lambda i: (0, i)),
                  pl.BlockSpec((_D, _RBLK), lambda i: (0, i)),
                  pl.BlockSpec((_D, _RBLK), lambda i: (0, i))],
        out_specs=[pl.BlockSpec((_RBLK // 8, 128), lambda i: (i, 0)),
                   pl.BlockSpec((_RBLK // 4, 128), lambda i: (i, 0))],
        out_shape=[jax.ShapeDtypeStruct((_WH_LINES, 128), jnp.float32),
                   jax.ShapeDtypeStruct((_CT_LINES, 128), jnp.float32)],
    )(Wh.T, Wc.T, Wt.T)


def _softmax_body(u_ref, len_ref, o_ref):
    u = u_ref[...]
    ln = len_ref[...]
    col = lax.broadcasted_iota(jnp.int32, u.shape, 1)
    u = jnp.where((col >= ln) | (col >= _C), -jnp.inf, u)
    m = jnp.max(u, axis=1, keepdims=True)
    sh = u - m
    lse = jnp.log(jnp.sum(jnp.exp(sh), axis=1, keepdims=True))
    o_ref[...] = (sh - lse)[:, :_C]


_BLK = 2048


def _tc_logsoftmax(util, lens2d):
    return pl.pallas_call(
        _softmax_body,
        grid=(_B // _BLK,),
        in_specs=[
            pl.BlockSpec((_BLK, _OP), lambda i: (i, 0)),
            pl.BlockSpec((_BLK, 1), lambda i: (i, 0)),
        ],
        out_specs=pl.BlockSpec((_BLK, _C), lambda i: (i, 0)),
        out_shape=jax.ShapeDtypeStruct((_B, _C), jnp.float32),
    )(util, lens2d)


def kernel(histories, history_lengths, choice_sets, choice_set_lengths,
           Wh, Wc, Wt):
    del history_lengths
    hp = jnp.pad(histories, ((0, 0), (0, _HP - _H)), mode="edge")
    cp = jnp.pad(choice_sets, ((0, 0), (0, _CPD - _C)), mode="edge")
    hrow = ((hp >> _SH) * _RBLK + (hp & (_RBLK // 8 - 1)) * 8
            + ((hp >> (_SH - 3)) & 7)).reshape(-1)
    crow = ((cp >> _SH) * _RBLK + (cp & (_RBLK // 4 - 1)) * 4
            + ((cp >> (_SH - 2)) & 3)).reshape(-1)
    wh, wct = _repack_tables(Wh, Wc, Wt)
    whv = wh.reshape(_WH_LINES * 8, _D)
    wctv = wct.reshape(_CT_LINES * 4, 2 * _D)
    util = _sc_utilities(hrow, crow, whv, wctv).reshape(_B, _OP)
    return _tc_logsoftmax(util, choice_set_lengths.reshape(_B, 1))


# speedup vs baseline: 1.0019x; 1.0019x over previous
import functools

import jax
import jax.numpy as jnp
from jax import lax
from jax.experimental import pallas as pl
from jax.experimental.pallas import tpu as pltpu
from jax.experimental.pallas import tpu_sc as plsc

_D = 16
_B = 16384
_H = 50
_C = 20
_BETA = 0.5
_N = 1000001

_HP = 56
_CPD = 24
_OP = 32

_NC = 2
_NS = 16
_NW = _NC * _NS
_RPW = _B // _NW

_RBLK = 32768
_SH = _RBLK.bit_length() - 1
_G = (_N + _RBLK - 1) // _RBLK
_WH_LINES = _G * (_RBLK // 8)
_CT_LINES = _G * (_RBLK // 4)


def _sc_body(hrow_hbm, crow_hbm, wh_hbm, wct_hbm,
             out_hbm,
             hg_v, cg_v, out_v,
             hb0, cb0, hb1, cb1,
             hs0, cs0, hs1, cs1):
    wid = lax.axis_index("s") * _NC + lax.axis_index("c")
    base = wid * _RPW

    pltpu.sync_copy(hrow_hbm.at[pl.ds(base * _HP, _RPW * _HP)], hg_v)
    pltpu.sync_copy(crow_hbm.at[pl.ds(base * _CPD, _RPW * _CPD)], cg_v)

    hbufs = (hb0, hb1)
    cbufs = (cb0, cb1)
    hsems = (hs0, hs1)
    csems = (cs0, cs1)

    _HG = _HP + _H
    _CG = _CPD + _C

    def issue(pair, b):
        pltpu.async_copy(
            wh_hbm.at[hg_v.at[pl.ds(pair * 2 * _HP, _HG)]],
            hbufs[b], hsems[b])
        pltpu.async_copy(
            wct_hbm.at[cg_v.at[pl.ds(pair * 2 * _CPD, _CG)]],
            cbufs[b], csems[b])

    def wait(pair, b):
        pltpu.make_async_copy(
            wh_hbm.at[hg_v.at[pl.ds(pair * 2 * _HP, _HG)]],
            hbufs[b], hsems[b]).wait()
        pltpu.make_async_copy(
            wct_hbm.at[cg_v.at[pl.ds(pair * 2 * _CPD, _CG)]],
            cbufs[b], csems[b]).wait()

    lanes = lax.iota(jnp.int32, _D)

    def compute(row, b, ho, co):
        hb = hbufs[b]
        cb = cbufs[b]
        accs = [hb[ho], hb[ho + 1] * _BETA,
                hb[ho + 2] * _BETA ** 2, hb[ho + 3] * _BETA ** 3]
        for h in range(4, _H):
            accs[h % 4] = accs[h % 4] + hb[ho + h] * (_BETA ** h)
        acc = (accs[0] + accs[1]) + (accs[2] + accs[3])
        ctx = [cb[co + c, 0:_D] for c in range(_C)]
        ss = [ctx[0], ctx[1], ctx[2], ctx[3]]
        for c in range(4, _C):
            ss[c % 4] = ss[c % 4] + ctx[c]
        s = (ss[0] + ss[1]) + (ss[2] + ss[3])
        a = acc + s
        lo = jnp.zeros((_D,), jnp.float32)
        hi = jnp.zeros((_D,), jnp.float32)
        for c in range(_C):
            tgt = cb[co + c, _D:2 * _D]
            u = jnp.sum(tgt * (a - ctx[c]))
            if c < _D:
                lo = jnp.where(lanes == c, u, lo)
            else:
                hi = jnp.where(lanes == (c - _D), u, hi)
        out_v[pl.ds(row * _OP, _D)] = lo
        out_v[pl.ds(row * _OP + _D, _D)] = hi

    issue(0, 0)

    def body(i, carry):
        p = i * 2
        for b in range(2):
            pair = p + b
            nxt = pair + 1

            @pl.when(nxt < _RPW // 2)
            def _():
                issue(nxt, 1 - b)

            wait(pair, b)
            compute(pair * 2, b, 0, 0)
            compute(pair * 2 + 1, b, _HP, _CPD)
        return carry

    lax.fori_loop(0, _RPW // 4, body, 0, unroll=False)

    pltpu.sync_copy(out_v, out_hbm.at[pl.ds(base * _OP, _RPW * _OP)])


_sc_utilities = functools.partial(
    pl.kernel,
    out_type=jax.ShapeDtypeStruct((_B * _OP,), jnp.float32),
    mesh=plsc.VectorSubcoreMesh(core_axis_name="c", subcore_axis_name="s"),
    compiler_params=pltpu.CompilerParams(
        needs_layout_passes=False, use_tc_tiling_on_sc=False),
    scratch_types=[
        pltpu.VMEM((_RPW * _HP,), jnp.int32),
        pltpu.VMEM((_RPW * _CPD,), jnp.int32),
        pltpu.VMEM((_RPW * _OP,), jnp.float32),
        pltpu.VMEM((_HP + _H, _D), jnp.float32),
        pltpu.VMEM((_CPD + _C, 2 * _D), jnp.float32),
        pltpu.VMEM((_HP + _H, _D), jnp.float32),
        pltpu.VMEM((_CPD + _C, 2 * _D), jnp.float32),
        pltpu.SemaphoreType.DMA,
        pltpu.SemaphoreType.DMA,
        pltpu.SemaphoreType.DMA,
        pltpu.SemaphoreType.DMA,
    ],
)(_sc_body)


def _mxu_t(x):
    eye = jnp.eye(128, dtype=jnp.float32)
    return lax.dot_general(x, eye, (((0,), (0,)), ((), ())),
                           preferred_element_type=jnp.float32)


def _repack_body(ht_ref, ct_ref, tt_ref, oh_ref, oct_ref):
    xh = ht_ref[...]
    w8 = _RBLK // 8
    out2h = jnp.concatenate(
        [xh[:, k * w8:(k + 1) * w8] for k in range(8)], axis=0)
    oh_ref[...] = _mxu_t(out2h)
    xc = ct_ref[...]
    xt = tt_ref[...]
    w4 = _RBLK // 4
    parts = []
    for k in range(4):
        parts.append(xc[:, k * w4:(k + 1) * w4])
        parts.append(xt[:, k * w4:(k + 1) * w4])
    out2c = jnp.concatenate(parts, axis=0)
    oct_ref[...] = _mxu_t(out2c)


def _repack_tables(Wh, Wc, Wt):
    return pl.pallas_call(
        _repack_body,
        grid=(_G,),
        in_specs=[pl.BlockSpec((_D, _RBLK), lambda i: (0, i)),
                  pl.BlockSpec((_D, _RBLK), lambda i: (0, i)),
                  pl.BlockSpec((_D, _RBLK), lambda i: (0, i))],
        out_specs=[pl.BlockSpec((_RBLK // 8, 128), lambda i: (i, 0)),
                   pl.BlockSpec((_RBLK // 4, 128), lambda i: (i, 0))],
        out_shape=[jax.ShapeDtypeStruct((_WH_LINES, 128), jnp.float32),
                   jax.ShapeDtypeStruct((_CT_LINES, 128), jnp.float32)],
    )(Wh.T, Wc.T, Wt.T)


def _softmax_body(u_ref, len_ref, o_ref):
    u = u_ref[...]
    ln = len_ref[...]
    col = lax.broadcasted_iota(jnp.int32, u.shape, 1)
    u = jnp.where((col >= ln) | (col >= _C), -jnp.inf, u)
    m = jnp.max(u, axis=1, keepdims=True)
    sh = u - m
    lse = jnp.log(jnp.sum(jnp.exp(sh), axis=1, keepdims=True))
    o_ref[...] = (sh - lse)[:, :_C]


_BLK = 2048


def _tc_logsoftmax(util, lens2d):
    return pl.pallas_call(
        _softmax_body,
        grid=(_B // _BLK,),
        in_specs=[
            pl.BlockSpec((_BLK, _OP), lambda i: (i, 0)),
            pl.BlockSpec((_BLK, 1), lambda i: (i, 0)),
        ],
        out_specs=pl.BlockSpec((_BLK, _C), lambda i: (i, 0)),
        out_shape=jax.ShapeDtypeStruct((_B, _C), jnp.float32),
    )(util, lens2d)


def kernel(histories, history_lengths, choice_sets, choice_set_lengths,
           Wh, Wc, Wt):
    del history_lengths
    hp = jnp.pad(histories, ((0, 0), (0, _HP - _H)), mode="edge")
    cp = jnp.pad(choice_sets, ((0, 0), (0, _CPD - _C)), mode="edge")
    hrow = ((hp >> _SH) * _RBLK + (hp & (_RBLK // 8 - 1)) * 8
            + ((hp >> (_SH - 3)) & 7)).reshape(-1)
    crow = ((cp >> _SH) * _RBLK + (cp & (_RBLK // 4 - 1)) * 4
            + ((cp >> (_SH - 2)) & 3)).reshape(-1)
    wh, wct = _repack_tables(Wh, Wc, Wt)
    whv = wh.reshape(_WH_LINES * 8, _D)
    wctv = wct.reshape(_CT_LINES * 4, 2 * _D)
    util = _sc_utilities(hrow, crow, whv, wctv).reshape(_B, _OP)
    return _tc_logsoftmax(util, choice_set_lengths.reshape(_B, 1))
